# Initial kernel scaffold; baseline (speedup 1.0000x reference)
import functools

import jax
import jax.numpy as jnp
from jax import lax
from jax.experimental import pallas as pl
from jax.experimental.pallas import tpu as pltpu
from jax.experimental.pallas import tpu_sc as plsc

NC = 2
NS = 16
NW = NC * NS
BLK = 128
BM = 1024


def _pad_to(n, m):
    return (n + m - 1) // m * m


def _make_deg_kernel(npad, nblk):
    mesh = plsc.VectorSubcoreMesh(core_axis_name="c", subcore_axis_name="s")
    rows_pt = npad // NS

    @functools.partial(
        pl.kernel,
        out_type=jax.ShapeDtypeStruct((NC, npad, 16), jnp.float32),
        mesh=mesh,
        scratch_types=[
            pltpu.VMEM_SHARED((npad, 16), jnp.float32),
            pltpu.VMEM((nblk, BLK), jnp.int32),
            pltpu.VMEM((BLK, 16), jnp.float32),
        ],
    )
    def deg_kernel(dst_hbm, zeros_hbm, out_hbm, deg_sh, dstv, ones_v):
        c = lax.axis_index("c")
        s = lax.axis_index("s")
        wid = c * NS + s
        r0 = s * rows_pt
        pltpu.sync_copy(zeros_hbm.at[pl.ds(r0, rows_pt)],
                        deg_sh.at[pl.ds(r0, rows_pt)])
        pltpu.sync_copy(dst_hbm.at[wid], dstv)

        @pl.loop(0, BLK)
        def _(i):
            ones_v[i, :] = jnp.ones((16,), jnp.float32)

        plsc.subcore_barrier()

        @pl.loop(0, nblk)
        def _(b):
            pltpu.sync_copy(ones_v, deg_sh.at[dstv.at[b]], add=True)

        plsc.subcore_barrier()
        pltpu.sync_copy(deg_sh.at[pl.ds(r0, rows_pt)],
                        out_hbm.at[c, pl.ds(r0, rows_pt)])

    return deg_kernel


def _make_accum_kernel(npad, nblk, d):
    mesh = plsc.VectorSubcoreMesh(core_axis_name="c", subcore_axis_name="s")
    rows_pt = npad // NS

    @functools.partial(
        pl.kernel,
        out_type=jax.ShapeDtypeStruct((NC, npad, d), jnp.float32),
        mesh=mesh,
        scratch_types=[
            pltpu.VMEM_SHARED((npad, d), jnp.float32),
            pltpu.VMEM((nblk, BLK), jnp.int32),
            pltpu.VMEM((nblk, BLK), jnp.int32),
            pltpu.VMEM((BLK, d), jnp.float32),
        ],
    )
    def accum_kernel(y_hbm, src_hbm, dst_hbm, zeros_hbm, out_hbm,
                     acc_sh, srcv, dstv, rows):
        c = lax.axis_index("c")
        s = lax.axis_index("s")
        wid = c * NS + s
        r0 = s * rows_pt
        pltpu.sync_copy(zeros_hbm.at[pl.ds(r0, rows_pt)],
                        acc_sh.at[pl.ds(r0, rows_pt)])
        pltpu.sync_copy(src_hbm.at[wid], srcv)
        pltpu.sync_copy(dst_hbm.at[wid], dstv)
        plsc.subcore_barrier()

        @pl.loop(0, nblk)
        def _(b):
            pltpu.sync_copy(y_hbm.at[srcv.at[b]], rows)
            pltpu.sync_copy(rows, acc_sh.at[dstv.at[b]], add=True)

        plsc.subcore_barrier()
        pltpu.sync_copy(acc_sh.at[pl.ds(r0, rows_pt)],
                        out_hbm.at[c, pl.ds(r0, rows_pt)])

    return accum_kernel


def _matmul(x, w):
    m, k = x.shape
    _, n = w.shape

    def body(x_ref, w_ref, o_ref):
        o_ref[...] = jnp.dot(x_ref[...], w_ref[...],
                             preferred_element_type=jnp.float32)

    return pl.pallas_call(
        body,
        grid=(m // BM,),
        in_specs=[pl.BlockSpec((BM, k), lambda i: (i, 0)),
                  pl.BlockSpec((k, n), lambda i: (0, 0))],
        out_specs=pl.BlockSpec((BM, n), lambda i: (i, 0)),
        out_shape=jax.ShapeDtypeStruct((m, n), jnp.float32),
    )(x, w)


def _prescale(xw, d0, d1):
    m, n = xw.shape

    def body(xw_ref, d0_ref, d1_ref, o_ref):
        deg = d0_ref[:, :1] + d1_ref[:, :1] + 1.0
        o_ref[...] = lax.rsqrt(deg) * xw_ref[...]

    return pl.pallas_call(
        body,
        grid=(m // BM,),
        in_specs=[pl.BlockSpec((BM, n), lambda i: (i, 0)),
                  pl.BlockSpec((BM, 16), lambda i: (i, 0)),
                  pl.BlockSpec((BM, 16), lambda i: (i, 0))],
        out_specs=pl.BlockSpec((BM, n), lambda i: (i, 0)),
        out_shape=jax.ShapeDtypeStruct((m, n), jnp.float32),
    )(xw, d0, d1)


def _layer2(a0, a1, y1, d0, d1, b1, w2):
    m, n = y1.shape
    _, n2 = w2.shape

    def body(a0_ref, a1_ref, y1_ref, d0_ref, d1_ref, b1_ref, w2_ref, o_ref):
        deg = d0_ref[:, :1] + d1_ref[:, :1] + 1.0
        dis = lax.rsqrt(deg)
        h = dis * (a0_ref[...] + a1_ref[...] + y1_ref[...]) + b1_ref[...]
        h = jnp.maximum(h, 0.0)
        o_ref[...] = dis * jnp.dot(h, w2_ref[...],
                                   preferred_element_type=jnp.float32)

    return pl.pallas_call(
        body,
        grid=(m // BM,),
        in_specs=[pl.BlockSpec((BM, n), lambda i: (i, 0)),
                  pl.BlockSpec((BM, n), lambda i: (i, 0)),
                  pl.BlockSpec((BM, n), lambda i: (i, 0)),
                  pl.BlockSpec((BM, 16), lambda i: (i, 0)),
                  pl.BlockSpec((BM, 16), lambda i: (i, 0)),
                  pl.BlockSpec((1, n), lambda i: (0, 0)),
                  pl.BlockSpec((n, n2), lambda i: (0, 0))],
        out_specs=pl.BlockSpec((BM, n2), lambda i: (i, 0)),
        out_shape=jax.ShapeDtypeStruct((m, n2), jnp.float32),
    )(a0, a1, y1, d0, d1, b1, w2)


def _final(a0, a1, y2, d0, d1, b2):
    m, n = y2.shape

    def body(a0_ref, a1_ref, y2_ref, d0_ref, d1_ref, b2_ref, o_ref):
        deg = d0_ref[:, :1] + d1_ref[:, :1] + 1.0
        dis = lax.rsqrt(deg)
        o_ref[...] = dis * (a0_ref[...] + a1_ref[...] + y2_ref[...]) \
            + b2_ref[...]

    return pl.pallas_call(
        body,
        grid=(m // BM,),
        in_specs=[pl.BlockSpec((BM, n), lambda i: (i, 0)),
                  pl.BlockSpec((BM, n), lambda i: (i, 0)),
                  pl.BlockSpec((BM, n), lambda i: (i, 0)),
                  pl.BlockSpec((BM, 16), lambda i: (i, 0)),
                  pl.BlockSpec((BM, 16), lambda i: (i, 0)),
                  pl.BlockSpec((1, n), lambda i: (0, 0))],
        out_specs=pl.BlockSpec((BM, n), lambda i: (i, 0)),
        out_shape=jax.ShapeDtypeStruct((m, n), jnp.float32),
    )(a0, a1, y2, d0, d1, b2)


def kernel(x, W1, b1, W2, b2, edge_index):
    n, d_in = x.shape
    d_hid = W1.shape[1]
    d_out = W2.shape[1]
    e = edge_index.shape[1]

    npad = _pad_to(n + 1, NS * BM // jnp.gcd(NS, BM).item() if False else 1280)
    epad = _pad_to(e, NW * BLK)
    nblk = epad // (NW * BLK)

    src = edge_index[0].astype(jnp.int32)
    dst = edge_index[1].astype(jnp.int32)
    if epad > e:
        pad = jnp.full((epad - e,), n, jnp.int32)
        src = jnp.concatenate([src, pad])
        dst = jnp.concatenate([dst, pad])
    src = src.reshape(NW, nblk, BLK)
    dst = dst.reshape(NW, nblk, BLK)

    x_pad = jnp.concatenate(
        [x, jnp.zeros((npad - n, d_in), jnp.float32)], axis=0)
    z16 = jnp.zeros((npad, 16), jnp.float32)
    zhid = jnp.zeros((npad, d_hid), jnp.float32)
    zout = jnp.zeros((npad, d_out), jnp.float32)
    b1r = b1.reshape(1, d_hid)
    b2r = b2.reshape(1, d_out)

    dp = _make_deg_kernel(npad, nblk)(dst, z16)
    d0, d1 = dp[0], dp[1]

    xw1 = _matmul(x_pad, W1)
    y1 = _prescale(xw1, d0, d1)

    a1 = _make_accum_kernel(npad, nblk, d_hid)(y1, src, dst, zhid)
    y2 = _layer2(a1[0], a1[1], y1, d0, d1, b1r, W2)

    a2 = _make_accum_kernel(npad, nblk, d_out)(y2, src, dst, zout)
    out = _final(a2[0], a2[1], y2, d0, d1, b2r)
    return out[:n]


# speedup vs baseline: 11.8948x; 11.8948x over previous
import functools

import jax
import jax.numpy as jnp
from jax import lax
from jax.experimental import pallas as pl
from jax.experimental.pallas import tpu as pltpu
from jax.experimental.pallas import tpu_sc as plsc

NC = 2
NS = 16
NW = NC * NS
BLK = 128
BM = 1024


def _pad_to(n, m):
    return (n + m - 1) // m * m


def _make_deg_kernel(npad, nblk):
    mesh = plsc.VectorSubcoreMesh(core_axis_name="c", subcore_axis_name="s", num_cores=NC, num_subcores=NS)
    rows_pt = npad // NS

    @functools.partial(
        pl.kernel,
        out_type=jax.ShapeDtypeStruct((NC, npad, 16), jnp.float32),
        mesh=mesh,
        scratch_types=[
            pltpu.VMEM_SHARED((npad, 16), jnp.float32),
            pltpu.VMEM((nblk, BLK), jnp.int32),
            pltpu.VMEM((BLK, 16), jnp.float32),
        ],
    )
    def deg_kernel(dst_hbm, zeros_hbm, out_hbm, deg_sh, dstv, ones_v):
        c = lax.axis_index("c")
        s = lax.axis_index("s")
        wid = c * NS + s
        r0 = s * rows_pt
        pltpu.sync_copy(zeros_hbm.at[pl.ds(r0, rows_pt)],
                        deg_sh.at[pl.ds(r0, rows_pt)])
        pltpu.sync_copy(dst_hbm.at[wid], dstv)

        @pl.loop(0, BLK)
        def _(i):
            ones_v[i, :] = jnp.ones((16,), jnp.float32)

        plsc.subcore_barrier()

        @pl.loop(0, nblk)
        def _(b):
            pltpu.sync_copy(ones_v, deg_sh.at[dstv.at[b]], add=True)

        plsc.subcore_barrier()
        pltpu.sync_copy(deg_sh.at[pl.ds(r0, rows_pt)],
                        out_hbm.at[c, pl.ds(r0, rows_pt)])

    return deg_kernel


def _make_accum_kernel(npad, nblk, d):
    mesh = plsc.VectorSubcoreMesh(core_axis_name="c", subcore_axis_name="s", num_cores=NC, num_subcores=NS)
    rows_pt = npad // NS

    @functools.partial(
        pl.kernel,
        out_type=jax.ShapeDtypeStruct((NC, npad, d), jnp.float32),
        mesh=mesh,
        scratch_types=[
            pltpu.VMEM_SHARED((npad, d), jnp.float32),
            pltpu.VMEM((nblk, BLK), jnp.int32),
            pltpu.VMEM((nblk, BLK), jnp.int32),
            pltpu.VMEM((BLK, d), jnp.float32),
        ],
    )
    def accum_kernel(y_hbm, src_hbm, dst_hbm, zeros_hbm, out_hbm,
                     acc_sh, srcv, dstv, rows):
        c = lax.axis_index("c")
        s = lax.axis_index("s")
        wid = c * NS + s
        r0 = s * rows_pt
        pltpu.sync_copy(zeros_hbm.at[pl.ds(r0, rows_pt)],
                        acc_sh.at[pl.ds(r0, rows_pt)])
        pltpu.sync_copy(src_hbm.at[wid], srcv)
        pltpu.sync_copy(dst_hbm.at[wid], dstv)
        plsc.subcore_barrier()

        @pl.loop(0, nblk)
        def _(b):
            pltpu.sync_copy(y_hbm.at[srcv.at[b]], rows)
            pltpu.sync_copy(rows, acc_sh.at[dstv.at[b]], add=True)

        plsc.subcore_barrier()
        pltpu.sync_copy(acc_sh.at[pl.ds(r0, rows_pt)],
                        out_hbm.at[c, pl.ds(r0, rows_pt)])

    return accum_kernel


def _matmul(x, w):
    m, k = x.shape
    _, n = w.shape

    def body(x_ref, w_ref, o_ref):
        o_ref[...] = jnp.dot(x_ref[...], w_ref[...],
                             preferred_element_type=jnp.float32)

    return pl.pallas_call(
        body,
        grid=(m // BM,),
        in_specs=[pl.BlockSpec((BM, k), lambda i: (i, 0)),
                  pl.BlockSpec((k, n), lambda i: (0, 0))],
        out_specs=pl.BlockSpec((BM, n), lambda i: (i, 0)),
        out_shape=jax.ShapeDtypeStruct((m, n), jnp.float32),
    )(x, w)


def _prescale(xw, d0, d1):
    m, n = xw.shape

    def body(xw_ref, d0_ref, d1_ref, o_ref):
        deg = d0_ref[:, :1] + d1_ref[:, :1] + 1.0
        o_ref[...] = lax.rsqrt(deg) * xw_ref[...]

    return pl.pallas_call(
        body,
        grid=(m // BM,),
        in_specs=[pl.BlockSpec((BM, n), lambda i: (i, 0)),
                  pl.BlockSpec((BM, 16), lambda i: (i, 0)),
                  pl.BlockSpec((BM, 16), lambda i: (i, 0))],
        out_specs=pl.BlockSpec((BM, n), lambda i: (i, 0)),
        out_shape=jax.ShapeDtypeStruct((m, n), jnp.float32),
    )(xw, d0, d1)


def _layer2(a0, a1, y1, d0, d1, b1, w2):
    m, n = y1.shape
    _, n2 = w2.shape

    def body(a0_ref, a1_ref, y1_ref, d0_ref, d1_ref, b1_ref, w2_ref, o_ref):
        deg = d0_ref[:, :1] + d1_ref[:, :1] + 1.0
        dis = lax.rsqrt(deg)
        h = dis * (a0_ref[...] + a1_ref[...] + y1_ref[...]) + b1_ref[...]
        h = jnp.maximum(h, 0.0)
        o_ref[...] = dis * jnp.dot(h, w2_ref[...],
                                   preferred_element_type=jnp.float32)

    return pl.pallas_call(
        body,
        grid=(m // BM,),
        in_specs=[pl.BlockSpec((BM, n), lambda i: (i, 0)),
                  pl.BlockSpec((BM, n), lambda i: (i, 0)),
                  pl.BlockSpec((BM, n), lambda i: (i, 0)),
                  pl.BlockSpec((BM, 16), lambda i: (i, 0)),
                  pl.BlockSpec((BM, 16), lambda i: (i, 0)),
                  pl.BlockSpec((1, n), lambda i: (0, 0)),
                  pl.BlockSpec((n, n2), lambda i: (0, 0))],
        out_specs=pl.BlockSpec((BM, n2), lambda i: (i, 0)),
        out_shape=jax.ShapeDtypeStruct((m, n2), jnp.float32),
    )(a0, a1, y1, d0, d1, b1, w2)


def _final(a0, a1, y2, d0, d1, b2):
    m, n = y2.shape

    def body(a0_ref, a1_ref, y2_ref, d0_ref, d1_ref, b2_ref, o_ref):
        deg = d0_ref[:, :1] + d1_ref[:, :1] + 1.0
        dis = lax.rsqrt(deg)
        o_ref[...] = dis * (a0_ref[...] + a1_ref[...] + y2_ref[...]) \
            + b2_ref[...]

    return pl.pallas_call(
        body,
        grid=(m // BM,),
        in_specs=[pl.BlockSpec((BM, n), lambda i: (i, 0)),
                  pl.BlockSpec((BM, n), lambda i: (i, 0)),
                  pl.BlockSpec((BM, n), lambda i: (i, 0)),
                  pl.BlockSpec((BM, 16), lambda i: (i, 0)),
                  pl.BlockSpec((BM, 16), lambda i: (i, 0)),
                  pl.BlockSpec((1, n), lambda i: (0, 0))],
        out_specs=pl.BlockSpec((BM, n), lambda i: (i, 0)),
        out_shape=jax.ShapeDtypeStruct((m, n), jnp.float32),
    )(a0, a1, y2, d0, d1, b2)


def kernel(x, W1, b1, W2, b2, edge_index):
    n, d_in = x.shape
    d_hid = W1.shape[1]
    d_out = W2.shape[1]
    e = edge_index.shape[1]

    assert BM % NS == 0
    npad = _pad_to(n + 1, BM)
    epad = _pad_to(e, NW * BLK)
    nblk = epad // (NW * BLK)

    src = edge_index[0].astype(jnp.int32)
    dst = edge_index[1].astype(jnp.int32)
    if epad > e:
        pad = jnp.full((epad - e,), n, jnp.int32)
        src = jnp.concatenate([src, pad])
        dst = jnp.concatenate([dst, pad])
    src = src.reshape(NW, nblk, BLK)
    dst = dst.reshape(NW, nblk, BLK)

    d_msg = _pad_to(d_out, 128)
    W2p = jnp.concatenate(
        [W2, jnp.zeros((d_hid, d_msg - d_out), jnp.float32)], axis=1)
    b2p = jnp.concatenate([b2, jnp.zeros((d_msg - d_out,), jnp.float32)])

    x_pad = jnp.concatenate(
        [x, jnp.zeros((npad - n, d_in), jnp.float32)], axis=0)
    z16 = jnp.zeros((npad, 16), jnp.float32)
    zhid = jnp.zeros((npad, d_hid), jnp.float32)
    zout = jnp.zeros((npad, d_msg), jnp.float32)
    b1r = b1.reshape(1, d_hid)
    b2r = b2p.reshape(1, d_msg)

    dp = _make_deg_kernel(npad, nblk)(dst, z16)
    d0, d1 = dp[0], dp[1]

    xw1 = _matmul(x_pad, W1)
    y1 = _prescale(xw1, d0, d1)

    a1 = _make_accum_kernel(npad, nblk, d_hid)(y1, src, dst, zhid)
    y2 = _layer2(a1[0], a1[1], y1, d0, d1, b1r, W2p)

    a2 = _make_accum_kernel(npad, nblk, d_msg)(y2, src, dst, zout)
    out = _final(a2[0], a2[1], y2, d0, d1, b2r)
    return out[:n, :d_out]
